# baseline (speedup 1.0000x reference)
import functools

import jax
import jax.numpy as jnp
from jax import lax
from jax.experimental import pallas as pl
from jax.experimental.pallas import tpu as pltpu
from jax.experimental.pallas import tpu_sc as plsc

NW = 32
HALF = 100
NBUF = 4


def _emb_kernel(B, T, D, n_seq):
  mesh = plsc.VectorSubcoreMesh(
      core_axis_name="c", subcore_axis_name="s", num_cores=2, num_subcores=16)
  n_rounds = n_seq // NBUF

  @functools.partial(
      pl.kernel,
      mesh=mesh,
      out_type=jax.ShapeDtypeStruct((B, T, D), jnp.float32),
      scratch_types=[
          pltpu.VMEM((n_seq, 2, HALF), jnp.int32),
          pltpu.VMEM_SHARED((T, D), jnp.float32),
          [pltpu.VMEM((T, D), jnp.float32)] * NBUF,
          pltpu.SemaphoreType.DMA((NBUF,)),
          pltpu.SemaphoreType.DMA((NBUF,)),
          pltpu.SemaphoreType.DMA((NBUF,)),
      ],
  )
  def k(idx_hbm, tab_hbm, pos_hbm, out_hbm,
        idx_v, pos_sh, accs, ssem, gsem, osem):
    sid = lax.axis_index("s")
    wid = lax.axis_index("c") * 16 + sid
    base = wid * n_seq

    pltpu.sync_copy(idx_hbm.at[wid], idx_v)
    @pl.when(sid == 0)
    def _():
      pltpu.sync_copy(pos_hbm, pos_sh)
    plsc.subcore_barrier()

    def seed(b):
      pltpu.async_copy(pos_sh, accs[b], ssem.at[b])

    def seed_wait(b):
      pltpu.make_async_copy(pos_sh, accs[b], ssem.at[b]).wait()

    def gather_start(c, b):
      pltpu.async_copy(tab_hbm.at[idx_v.at[c, 0]],
                       accs[b].at[pl.ds(0, HALF)], gsem.at[b], add=True)
      pltpu.async_copy(tab_hbm.at[idx_v.at[c, 1]],
                       accs[b].at[pl.ds(HALF, HALF)], gsem.at[b], add=True)

    def gather_wait(c, b):
      for h in range(2):
        pltpu.make_async_copy(tab_hbm.at[idx_v.at[c, h]],
                              accs[b].at[pl.ds(h * HALF, HALF)],
                              gsem.at[b]).wait()

    def out_wait(b):
      pltpu.make_async_copy(accs[b], out_hbm.at[base], osem.at[b]).wait()

    seed(0)
    seed(1)
    seed(2)
    seed_wait(0)
    gather_start(0, 0)

    def round_body(r, carry):
      for b in range(NBUF):
        c = NBUF * r + b
        b1, b2 = (b + 1) % NBUF, (b + 2) % NBUF
        gather_wait(c, b)
        pltpu.async_copy(accs[b], out_hbm.at[base + c], osem.at[b])
        if b == NBUF - 1:
          @pl.when(r < n_rounds - 1)
          def _():
            seed_wait(b1)
            gather_start(c + 1, b1)
        else:
          seed_wait(b1)
          gather_start(c + 1, b1)
        if b < 2:
          @pl.when(r > 0)
          def _():
            out_wait(b2)
          if b == 1:
            seed(b2)
          else:
            @pl.when(r > 0)
            def _():
              seed(b2)
        else:
          out_wait(b2)

          @pl.when(c + 2 < n_seq)
          def _():
            seed(b2)
      return carry

    lax.fori_loop(0, n_rounds, round_body, 0)

    for b in (NBUF - 2, NBUF - 1):
      out_wait(b)

  return k


def kernel(inputs, embbedL, embbedP):
  B, T = inputs.shape
  V, D = embbedL.shape
  assert B % NW == 0 and T == 2 * HALF
  n_seq = B // NW
  assert n_seq % NBUF == 0 and n_seq // NBUF >= 2

  idx = inputs.reshape(NW, n_seq, 2, HALF).astype(jnp.int32)
  return _emb_kernel(B, T, D, n_seq)(idx, embbedL, embbedP)


# speedup vs baseline: 1.0671x; 1.0671x over previous
import functools

import jax
import jax.numpy as jnp
from jax import lax
from jax.experimental import pallas as pl
from jax.experimental.pallas import tpu as pltpu
from jax.experimental.pallas import tpu_sc as plsc

NW = 32
PIECES = (104, 96)
PMAX = 104
NBUF = 4


def _emb_kernel(B, T, D, n_seq):
  mesh = plsc.VectorSubcoreMesh(
      core_axis_name="c", subcore_axis_name="s", num_cores=2, num_subcores=16)
  n_rounds = n_seq // NBUF

  @functools.partial(
      pl.kernel,
      mesh=mesh,
      out_type=jax.ShapeDtypeStruct((B, T, D), jnp.float32),
      scratch_types=[
          pltpu.VMEM((n_seq, 2, PMAX), jnp.int32),
          pltpu.VMEM_SHARED((T, D), jnp.float32),
          [pltpu.VMEM((T, D), jnp.float32)] * NBUF,
          pltpu.SemaphoreType.DMA((NBUF,)),
          pltpu.SemaphoreType.DMA((NBUF,)),
          pltpu.SemaphoreType.DMA((NBUF,)),
      ],
  )
  def k(idx_hbm, tab_hbm, pos_hbm, out_hbm,
        idx_v, pos_sh, accs, ssem, gsem, osem):
    sid = lax.axis_index("s")
    wid = lax.axis_index("c") * 16 + sid
    base = wid * n_seq

    pltpu.sync_copy(idx_hbm.at[wid], idx_v)
    @pl.when(sid == 0)
    def _():
      pltpu.sync_copy(pos_hbm, pos_sh)
    plsc.subcore_barrier()

    def seed(b):
      pltpu.async_copy(pos_sh, accs[b], ssem.at[b])

    def seed_wait(b):
      pltpu.make_async_copy(pos_sh, accs[b], ssem.at[b]).wait()

    def gather_start(c, b):
      pltpu.async_copy(tab_hbm.at[idx_v.at[c, 0, pl.ds(0, PIECES[0])]],
                       accs[b].at[pl.ds(0, PIECES[0])], gsem.at[b], add=True)
      pltpu.async_copy(tab_hbm.at[idx_v.at[c, 1, pl.ds(0, PIECES[1])]],
                       accs[b].at[pl.ds(PIECES[0], PIECES[1])],
                       gsem.at[b], add=True)

    def gather_wait_h(c, b, h):
      off = 0 if h == 0 else PIECES[0]
      pltpu.make_async_copy(tab_hbm.at[idx_v.at[c, h, pl.ds(0, PIECES[h])]],
                            accs[b].at[pl.ds(off, PIECES[h])],
                            gsem.at[b]).wait()

    def out_start_h(c, b, h):
      off = 0 if h == 0 else PIECES[0]
      pltpu.async_copy(accs[b].at[pl.ds(off, PIECES[h])],
                       out_hbm.at[base + c, pl.ds(off, PIECES[h])],
                       osem.at[b])

    def out_wait(b):
      for h in range(2):
        off = 0 if h == 0 else PIECES[0]
        pltpu.make_async_copy(accs[b].at[pl.ds(off, PIECES[h])],
                              out_hbm.at[base, pl.ds(off, PIECES[h])],
                              osem.at[b]).wait()

    seed(0)
    seed(1)
    seed(2)
    seed_wait(0)
    gather_start(0, 0)

    def round_body(r, carry):
      for b in range(NBUF):
        c = NBUF * r + b
        b1, b2 = (b + 1) % NBUF, (b + 2) % NBUF
        if b == NBUF - 1:
          @pl.when(r < n_rounds - 1)
          def _():
            seed_wait(b1)
            gather_start(c + 1, b1)
        else:
          seed_wait(b1)
          gather_start(c + 1, b1)
        gather_wait_h(c, b, 0)
        out_start_h(c, b, 0)
        gather_wait_h(c, b, 1)
        out_start_h(c, b, 1)
        if b < 2:
          @pl.when(r > 0)
          def _():
            out_wait(b2)
          if b == 1:
            seed(b2)
          else:
            @pl.when(r > 0)
            def _():
              seed(b2)
        else:
          out_wait(b2)

          @pl.when(c + 2 < n_seq)
          def _():
            seed(b2)
      return carry

    lax.fori_loop(0, n_rounds, round_body, 0)

    for b in (NBUF - 2, NBUF - 1):
      out_wait(b)

  return k


def kernel(inputs, embbedL, embbedP):
  B, T = inputs.shape
  V, D = embbedL.shape
  assert B % NW == 0 and T == sum(PIECES)
  n_seq = B // NW
  assert n_seq % NBUF == 0 and n_seq // NBUF >= 2

  flat = inputs.reshape(NW, n_seq, T).astype(jnp.int32)
  p0 = flat[:, :, :PIECES[0]]
  p1 = jnp.pad(flat[:, :, PIECES[0]:], ((0, 0), (0, 0), (0, PMAX - PIECES[1])))
  idx = jnp.stack([p0, p1], axis=2)
  return _emb_kernel(B, T, D, n_seq)(idx, embbedL, embbedP)
